# baseline (speedup 1.0000x reference)
import functools

import jax
import jax.numpy as jnp
from jax import lax
from jax.experimental import pallas as pl
from jax.experimental.pallas import tpu as pltpu
from jax.experimental.pallas import tpu_sc as plsc

NUM_CORES = 2
NUM_SUBCORES = 16
NUM_WORKERS = NUM_CORES * NUM_SUBCORES
LANES = 16
KC = 32


@jax.jit
def _pos_embed_add(x, table, pos):
    B, S, D = x.shape
    s_per_w = S // NUM_WORKERS
    n_chunks = s_per_w // KC

    @functools.partial(
        pl.kernel,
        out_type=jax.ShapeDtypeStruct((B, S, D), jnp.float32),
        mesh=plsc.VectorSubcoreMesh(
            core_axis_name="c", subcore_axis_name="s"
        ),
        scratch_types=[
            pltpu.VMEM((s_per_w,), jnp.int32),
            pltpu.VMEM((KC, D), jnp.float32),
            pltpu.VMEM((KC, D), jnp.float32),
            pltpu.SemaphoreType.DMA,
        ],
    )
    def body(x_hbm, table_hbm, pos_hbm, out_hbm, idx_v, ebuf, xbuf, sem):
        wid = lax.axis_index("s") * NUM_CORES + lax.axis_index("c")
        s0 = wid * s_per_w
        pltpu.sync_copy(pos_hbm.at[pl.ds(s0, s_per_w)], idx_v)
        for i in range(n_chunks):
            s = s0 + i * KC
            pltpu.async_copy(
                table_hbm.at[idx_v.at[pl.ds(i * KC, KC)]], ebuf, sem
            ).wait()
            for b in range(B):
                pltpu.sync_copy(x_hbm.at[b, pl.ds(s, KC), :], xbuf)

                @pl.loop(0, KC)
                def _(r):
                    for c in range(D // LANES):
                        sl = pl.ds(c * LANES, LANES)
                        plsc.addupdate(xbuf.at[r, sl], ebuf[r, sl])

                pltpu.sync_copy(xbuf, out_hbm.at[b, pl.ds(s, KC), :])

    return body(x, table, pos)


def kernel(x, table, positional_ids):
    B, S, D = x.shape
    pos = positional_ids[0, :S].astype(jnp.int32)
    return _pos_embed_add(x, table, pos)


# speedup vs baseline: 1.5984x; 1.5984x over previous
import functools

import jax
import jax.numpy as jnp
from jax import lax
from jax.experimental import pallas as pl
from jax.experimental.pallas import tpu as pltpu
from jax.experimental.pallas import tpu_sc as plsc

NUM_CORES = 2
NUM_SUBCORES = 16
NUM_WORKERS = NUM_CORES * NUM_SUBCORES
LANES = 16
KC = 8


@jax.jit
def _pos_embed_add(x, table, pos):
    B, S, D = x.shape
    s_per_w = S // NUM_WORKERS
    n_chunks = s_per_w // KC

    @functools.partial(
        pl.kernel,
        out_type=jax.ShapeDtypeStruct((B, S, D), jnp.float32),
        mesh=plsc.VectorSubcoreMesh(
            core_axis_name="c", subcore_axis_name="s"
        ),
        scratch_types=[
            pltpu.VMEM((s_per_w,), jnp.int32),
            pltpu.VMEM((2, KC, D), jnp.float32),
            pltpu.VMEM((2 * B, KC, D), jnp.float32),
            pltpu.SemaphoreType.DMA((2,)),
            pltpu.SemaphoreType.DMA((2 * B,)),
            pltpu.SemaphoreType.DMA((2 * B,)),
        ],
    )
    def body(x_hbm, table_hbm, pos_hbm, out_hbm, idx_v, ebufs, xbufs,
             gsem, isem, osem):
        wid = lax.axis_index("s") * NUM_CORES + lax.axis_index("c")
        s0 = wid * s_per_w
        pltpu.sync_copy(pos_hbm.at[pl.ds(s0, s_per_w)], idx_v)

        def gd(i):
            p = lax.rem(i, 2)
            return pltpu.make_async_copy(
                table_hbm.at[idx_v.at[pl.ds(i * KC, KC)]],
                ebufs.at[p], gsem.at[p])

        def ind(i, b):
            p = lax.rem(i, 2)
            return pltpu.make_async_copy(
                x_hbm.at[b, pl.ds(s0 + i * KC, KC), :],
                xbufs.at[p * B + b], isem.at[p * B + b])

        def outd(i, b):
            p = lax.rem(i, 2)
            return pltpu.make_async_copy(
                xbufs.at[p * B + b],
                out_hbm.at[b, pl.ds(s0 + i * KC, KC), :],
                osem.at[p * B + b])

        gd(0).start()
        for b in range(B):
            ind(0, b).start()

        @pl.loop(0, n_chunks)
        def _(i):
            par = lax.rem(i, 2)
            gd(i).wait()

            @pl.when(i + 1 < n_chunks)
            def _():
                gd(i + 1).start()

            for b in range(B):
                ind(i, b).wait()

            @pl.when(i >= 1)
            def _():
                for b in range(B):
                    outd(i - 1, b).wait()

            @pl.when(i + 1 < n_chunks)
            def _():
                for b in range(B):
                    ind(i + 1, b).start()

            @pl.loop(0, KC)
            def _(r):
                for c in range(D // LANES):
                    sl = pl.ds(c * LANES, LANES)
                    v = ebufs[par, r, sl]
                    for b in range(B):
                        plsc.addupdate(xbufs.at[par * B + b, r, sl], v)

            for b in range(B):
                outd(i, b).start()

        for b in range(B):
            outd(n_chunks - 1, b).wait()

    return body(x, table, pos)


def kernel(x, table, positional_ids):
    B, S, D = x.shape
    pos = positional_ids[0, :S].astype(jnp.int32)
    return _pos_embed_add(x, table, pos)


# speedup vs baseline: 1.7667x; 1.1053x over previous
import functools

import jax
import jax.numpy as jnp
from jax import lax
from jax.experimental import pallas as pl
from jax.experimental.pallas import tpu as pltpu
from jax.experimental.pallas import tpu_sc as plsc

NUM_CORES = 2
NUM_SUBCORES = 16
NUM_WORKERS = NUM_CORES * NUM_SUBCORES
LANES = 16
KC = 8
PAR = 3


@jax.jit
def _pos_embed_add(x, table, pos):
    B, S, D = x.shape
    s_per_w = S // NUM_WORKERS
    n_chunks = s_per_w // KC

    @functools.partial(
        pl.kernel,
        out_type=jax.ShapeDtypeStruct((B, S, D), jnp.float32),
        mesh=plsc.VectorSubcoreMesh(
            core_axis_name="c", subcore_axis_name="s"
        ),
        scratch_types=[
            pltpu.VMEM((s_per_w,), jnp.int32),
            pltpu.VMEM((PAR, KC, D), jnp.float32),
            pltpu.VMEM((PAR * B, KC, D), jnp.float32),
            pltpu.SemaphoreType.DMA((PAR,)),
            pltpu.SemaphoreType.DMA((PAR * B,)),
            pltpu.SemaphoreType.DMA((PAR * B,)),
        ],
    )
    def body(x_hbm, table_hbm, pos_hbm, out_hbm, idx_v, ebufs, xbufs,
             gsem, isem, osem):
        wid = lax.axis_index("s") * NUM_CORES + lax.axis_index("c")
        s0 = wid * s_per_w
        pltpu.sync_copy(pos_hbm.at[pl.ds(s0, s_per_w)], idx_v)

        def gd(i):
            p = lax.rem(i, PAR)
            return pltpu.make_async_copy(
                table_hbm.at[idx_v.at[pl.ds(i * KC, KC)]],
                ebufs.at[p], gsem.at[p])

        def ind(i, b):
            p = lax.rem(i, PAR)
            return pltpu.make_async_copy(
                x_hbm.at[b, pl.ds(s0 + i * KC, KC), :],
                xbufs.at[p * B + b], isem.at[p * B + b])

        def outd(i, b):
            p = lax.rem(i, PAR)
            return pltpu.make_async_copy(
                xbufs.at[p * B + b],
                out_hbm.at[b, pl.ds(s0 + i * KC, KC), :],
                osem.at[p * B + b])

        for j in range(PAR - 1):
            gd(j).start()
            for b in range(B):
                ind(j, b).start()

        @pl.loop(0, n_chunks)
        def _(i):
            par = lax.rem(i, PAR)
            gd(i).wait()

            @pl.when(i + PAR - 1 < n_chunks)
            def _():
                gd(i + PAR - 1).start()

            for b in range(B):
                ind(i, b).wait()

            @pl.loop(0, KC)
            def _(r):
                for c in range(D // LANES):
                    sl = pl.ds(c * LANES, LANES)
                    v = ebufs[par, r, sl]
                    for b in range(B):
                        plsc.addupdate(xbufs.at[par * B + b, r, sl], v)

            for b in range(B):
                outd(i, b).start()

            @pl.when(i + PAR - 1 < n_chunks)
            def _():
                @pl.when(i >= 1)
                def _():
                    for b in range(B):
                        outd(i - 1, b).wait()

                for b in range(B):
                    ind(i + PAR - 1, b).start()

        for j in range(max(0, n_chunks - PAR), n_chunks):
            for b in range(B):
                outd(j, b).wait()

    return body(x, table, pos)


def kernel(x, table, positional_ids):
    B, S, D = x.shape
    pos = positional_ids[0, :S].astype(jnp.int32)
    return _pos_embed_add(x, table, pos)
